# baseline (speedup 1.0000x reference)
import jax
import jax.numpy as jnp
from jax import lax
from jax.experimental import pallas as pl
from jax.experimental.pallas import tpu as pltpu
from jax.experimental.pallas import tpu_sc as plsc

N = 10000
E = 320000
D = 128

NS = 16
CHUNK = 128
RPT = 640
ACC = NS * RPT
LAST = N - 15 * RPT
ARPT = 632
ALAST = N - 15 * ARPT
RING = 3
CPW_A = 168
E_PAD_A = NS * CHUNK * CPW_A
NGRP = CPW_A // RING

BLK = 400
GRID = N // BLK


def _ceil_to(x, m):
    return (x + m - 1) // m * m


CPW = _ceil_to(-(-E // (NS * CHUNK)), 8)
E_PAD = NS * CHUNK * CPW


def _zero_tile_slice(shared, rows, sid):
    z = jnp.zeros((16,), jnp.float32)

    def zrow(i, _):
        for k in range(D // 16):
            rows[i, pl.ds(k * 16, 16)] = z
        return 0

    lax.fori_loop(0, CHUNK, zrow, 0)
    for k in range(RPT // CHUNK):
        pltpu.sync_copy(rows, shared.at[pl.ds(sid * RPT + k * CHUNK, CHUNK)])


def _copy_out_tile_slice(shared, out0, sid):
    @pl.when(sid < NS - 1)
    def _():
        pltpu.sync_copy(shared.at[pl.ds(sid * RPT, RPT)],
                        out0.at[pl.ds(sid * RPT, RPT)])

    @pl.when(sid == NS - 1)
    def _():
        pltpu.sync_copy(shared.at[pl.ds((NS - 1) * RPT, LAST)],
                        out0.at[pl.ds((NS - 1) * RPT, LAST)])


def _sc_agg_body(x_hbm, comb_hbm, out0, shared, idx, r0, r1, r2, gsem, ssem):
    rows = (r0, r1, r2)
    sid = lax.axis_index("s")

    z = jnp.zeros((16,), jnp.float32)

    def zrow(i, _):
        for k in range(D // 16):
            rows[0][i, pl.ds(k * 16, 16)] = z
        return 0

    lax.fori_loop(0, CHUNK, zrow, 0)
    for k in range(4):
        pltpu.sync_copy(rows[0],
                        shared.at[pl.ds(sid * ARPT + k * CHUNK, CHUNK)])

    @pl.when(sid < NS - 1)
    def _():
        pltpu.sync_copy(rows[0].at[pl.ds(0, ARPT - 4 * CHUNK)],
                        shared.at[pl.ds(sid * ARPT + 4 * CHUNK,
                                        ARPT - 4 * CHUNK)])

    @pl.when(sid == NS - 1)
    def _():
        pltpu.sync_copy(rows[0].at[pl.ds(0, ALAST - 4 * CHUNK)],
                        shared.at[pl.ds((NS - 1) * ARPT + 4 * CHUNK,
                                        ALAST - 4 * CHUNK)])

    plsc.subcore_barrier()

    def scatter_drain():
        for b in range(RING):
            pltpu.make_async_copy(rows[b], shared.at[idx.at[3]], ssem).wait()

    def group(g, _):
        @pl.when(g > 0)
        def _():
            scatter_drain()

        pltpu.sync_copy(comb_hbm.at[pl.ds((sid * NGRP + g) * 8, 8)], idx)
        for b in range(RING):
            pltpu.async_copy(x_hbm.at[idx.at[b]], rows[b], gsem)
        for b in range(RING):
            pltpu.make_async_copy(x_hbm.at[idx.at[0]], rows[b], gsem).wait()
        for b in range(RING):
            pltpu.async_copy(rows[b], shared.at[idx.at[3 + b]], ssem,
                             add=True)
        return 0

    lax.fori_loop(0, NGRP, group, 0)
    scatter_drain()
    plsc.subcore_barrier()

    @pl.when(sid < NS - 1)
    def _():
        pltpu.sync_copy(shared.at[pl.ds(sid * ARPT, ARPT)],
                        out0.at[pl.ds(sid * ARPT, ARPT)])

    @pl.when(sid == NS - 1)
    def _():
        pltpu.sync_copy(shared.at[pl.ds((NS - 1) * ARPT, ALAST)],
                        out0.at[pl.ds((NS - 1) * ARPT, ALAST)])


def _sc_agg(x, comb):
    mesh = plsc.VectorSubcoreMesh(core_axis_name="c", subcore_axis_name="s",
                                  num_cores=1)
    f = pl.kernel(
        _sc_agg_body,
        out_type=jax.ShapeDtypeStruct((N, D), jnp.float32),
        mesh=mesh,
        scratch_types=(
            [pltpu.VMEM_SHARED((N, D), jnp.float32),
             pltpu.VMEM((8, CHUNK), jnp.int32)]
            + [pltpu.VMEM((CHUNK, D), jnp.float32) for _ in range(RING)]
            + [pltpu.SemaphoreType.DMA for _ in range(2)]
        ),
    )
    return f(x, comb)


def _sc_cnt_body(dst_hbm, out0, shared, dstbuf, rows, ones, ssem):
    sid = lax.axis_index("s")
    half = CPW // 2

    _zero_tile_slice(shared, rows, sid)
    one = jnp.ones((16,), jnp.float32)

    def orow(i, _):
        for k in range(D // 16):
            ones[i, pl.ds(k * 16, 16)] = one
        return 0

    lax.fori_loop(0, CHUNK, orow, 0)
    plsc.subcore_barrier()

    def group(g, _):
        for k in range(8):
            pltpu.async_copy(ones, shared.at[dstbuf.at[8 * g + k]], ssem,
                             add=True)
        for k in range(8):
            pltpu.make_async_copy(ones, shared.at[dstbuf.at[8 * g]],
                                  ssem).wait()
        return 0

    for h in range(2):
        base = sid * CPW + h * half
        pltpu.sync_copy(dst_hbm.at[pl.ds(base, half)], dstbuf)
        lax.fori_loop(0, half // 8, group, 0)
    plsc.subcore_barrier()

    _copy_out_tile_slice(shared, out0, sid)


def _sc_cnt(dst2d):
    mesh = plsc.VectorSubcoreMesh(core_axis_name="c", subcore_axis_name="s",
                                  num_cores=1)
    f = pl.kernel(
        _sc_cnt_body,
        out_type=jax.ShapeDtypeStruct((N, D), jnp.float32),
        mesh=mesh,
        scratch_types=[
            pltpu.VMEM_SHARED((ACC, D), jnp.float32),
            pltpu.VMEM((CPW // 2, CHUNK), jnp.int32),
            pltpu.VMEM((CHUNK, D), jnp.float32),
            pltpu.VMEM((CHUNK, D), jnp.float32),
            pltpu.SemaphoreType.DMA,
        ],
    )
    return f(dst2d)


def _enc_body(x_ref, w0, b0, w1, b1, w2, b2, o_ref):
    h = jnp.maximum(jnp.dot(x_ref[...], w0[...],
                            preferred_element_type=jnp.float32) + b0[...], 0.0)
    h = jnp.maximum(jnp.dot(h, w1[...],
                            preferred_element_type=jnp.float32) + b1[...], 0.0)
    o_ref[...] = jnp.maximum(jnp.dot(h, w2[...],
                                     preferred_element_type=jnp.float32) + b2[...], 0.0)


def _tc_encoder(x, W0, b0, W1, b1, W2, b2):
    H = W0.shape[1]
    full = lambda s: pl.BlockSpec(s, lambda i: (0, 0))
    return pl.pallas_call(
        _enc_body,
        grid=(GRID,),
        in_specs=[
            pl.BlockSpec((BLK, D), lambda i: (i, 0)),
            full((D, H)), full((1, H)),
            full((H, H)), full((1, H)),
            full((H, D)), full((1, D)),
        ],
        out_specs=pl.BlockSpec((BLK, D), lambda i: (i, 0)),
        out_shape=jax.ShapeDtypeStruct((N, D), jnp.float32),
    )(x, W0, b0.reshape(1, -1), W1, b1.reshape(1, -1), W2, b2.reshape(1, -1))


def _conv_body(x_ref, p_ref, c_ref, wl, wr, b, o_ref):
    inv = 1.0 / jnp.maximum(c_ref[:, 0:1], 1.0)
    agg = p_ref[...] * inv
    o_ref[...] = jnp.maximum(
        jnp.dot(agg, wl[...], preferred_element_type=jnp.float32) + b[...]
        + jnp.dot(x_ref[...], wr[...], preferred_element_type=jnp.float32),
        0.0)


def _tc_conv(x, p, cnt, Wl, Wr, b):
    full = lambda s: pl.BlockSpec(s, lambda i: (0, 0))
    blk = lambda s: pl.BlockSpec(s, lambda i: (i, 0))
    return pl.pallas_call(
        _conv_body,
        grid=(GRID,),
        in_specs=[
            blk((BLK, D)), blk((BLK, D)), blk((BLK, D)),
            full((D, D)), full((D, D)), full((1, D)),
        ],
        out_specs=pl.BlockSpec((BLK, D), lambda i: (i, 0)),
        out_shape=jax.ShapeDtypeStruct((N, D), jnp.float32),
    )(x, p, cnt, Wl, Wr, b.reshape(1, -1))


def _ro_body(x_ref, ix_ref, w0a, w0b, b0, w1, b1, w2, b2, o_ref):
    h = jnp.maximum(
        jnp.dot(x_ref[...], w0a[...], preferred_element_type=jnp.float32)
        + jnp.dot(ix_ref[...], w0b[...], preferred_element_type=jnp.float32)
        + b0[...], 0.0)
    h = jnp.maximum(jnp.dot(h, w1[...],
                            preferred_element_type=jnp.float32) + b1[...], 0.0)
    o_ref[...] = jnp.dot(h, w2[...],
                         preferred_element_type=jnp.float32) + b2[...]


def _tc_readout(x, in_x, W0, b0, W1, b1, W2, b2):
    HR = W1.shape[0]
    full = lambda s: pl.BlockSpec(s, lambda i: (0, 0))
    blk = lambda s: pl.BlockSpec(s, lambda i: (i, 0))
    return pl.pallas_call(
        _ro_body,
        grid=(GRID,),
        in_specs=[
            blk((BLK, D)), blk((BLK, D)),
            full((D, HR)), full((D, HR)), full((1, HR)),
            full((HR, HR)), full((1, HR)),
            full((HR, D)), full((1, D)),
        ],
        out_specs=pl.BlockSpec((BLK, D), lambda i: (i, 0)),
        out_shape=jax.ShapeDtypeStruct((N, D), jnp.float32),
    )(x, in_x, W0[:D], W0[D:], b0.reshape(1, -1), W1, b1.reshape(1, -1),
      W2, b2.reshape(1, -1))


def kernel(in_x, edge_index, emb_W0, emb_b0, emb_W1, emb_b1, emb_W2, emb_b2,
           conv0_Wl, conv0_Wr, conv0_b, conv1_Wl, conv1_Wr, conv1_b,
           conv2_Wl, conv2_Wr, conv2_b, ro_W0, ro_b0, ro_W1, ro_b1,
           ro_W2, ro_b2):
    src = edge_index[0]
    dst = edge_index[1]
    npad = E_PAD - E
    dst2d_cnt = jnp.concatenate(
        [dst, jnp.full((npad,), N, jnp.int32)]).reshape(-1, CHUNK)
    npad_a = E_PAD_A - E
    srcT = jnp.concatenate(
        [src, jnp.full((npad_a,), N, jnp.int32)]).reshape(NS, NGRP, 3, CHUNK)
    dstT = jnp.concatenate(
        [dst, jnp.zeros((npad_a,), jnp.int32)]).reshape(NS, NGRP, 3, CHUNK)
    comb = jnp.concatenate(
        [srcT, dstT, jnp.zeros((NS, NGRP, 2, CHUNK), jnp.int32)],
        axis=2).reshape(-1, CHUNK)

    cnt = _sc_cnt(dst2d_cnt)
    x = _tc_encoder(in_x, emb_W0, emb_b0, emb_W1, emb_b1, emb_W2, emb_b2)
    for Wl, Wr, b in ((conv0_Wl, conv0_Wr, conv0_b),
                      (conv1_Wl, conv1_Wr, conv1_b),
                      (conv2_Wl, conv2_Wr, conv2_b)):
        xz = jnp.pad(x, ((0, 8), (0, 0)))
        p = _sc_agg(xz, comb)
        x = _tc_conv(x, p, cnt, Wl, Wr, b)
    return _tc_readout(x, in_x, ro_W0, ro_b0, ro_W1, ro_b1, ro_W2, ro_b2)


# speedup vs baseline: 1.8602x; 1.8602x over previous
import jax
import jax.numpy as jnp
from jax import lax
from jax.experimental import pallas as pl
from jax.experimental.pallas import tpu as pltpu
from jax.experimental.pallas import tpu_sc as plsc

N = 10000
E = 320000
D = 128

NS = 16
CHUNK = 128
RPT = 640
ACC = NS * RPT
LAST = N - 15 * RPT
Q = 40

BLK = 400
GRID = N // BLK


def _ceil_to(x, m):
    return (x + m - 1) // m * m


CPW = _ceil_to(-(-E // (NS * CHUNK)), 8)
E_PAD = NS * CHUNK * CPW


def _zero_tile_slice(shared, rows, sid):
    z = jnp.zeros((16,), jnp.float32)

    def zrow(i, _):
        for k in range(D // 16):
            rows[i, pl.ds(k * 16, 16)] = z
        return 0

    lax.fori_loop(0, CHUNK, zrow, 0)
    for k in range(RPT // CHUNK):
        pltpu.sync_copy(rows, shared.at[pl.ds(sid * RPT + k * CHUNK, CHUNK)])


def _copy_out_tile_slice(shared, out0, sid):
    @pl.when(sid < NS - 1)
    def _():
        pltpu.sync_copy(shared.at[pl.ds(sid * RPT, RPT)],
                        out0.at[pl.ds(sid * RPT, RPT)])

    @pl.when(sid == NS - 1)
    def _():
        pltpu.sync_copy(shared.at[pl.ds((NS - 1) * RPT, LAST)],
                        out0.at[pl.ds((NS - 1) * RPT, LAST)])


def _sc_agg_body(x_hbm, src_hbm, dst_hbm, out0, shared, srcbuf, dstbuf,
                 rows0, rows1, gs0, gs1, ss0, ss1):
    sid = lax.axis_index("s")
    rows = (rows0, rows1)
    gs = (gs0, gs1)
    ss = (ss0, ss1)

    _zero_tile_slice(shared, rows0, sid)
    plsc.subcore_barrier()

    for q in range(CPW // Q):
        base = sid * CPW + q * Q
        pltpu.sync_copy(src_hbm.at[pl.ds(base, Q)], srcbuf)
        pltpu.sync_copy(dst_hbm.at[pl.ds(base, Q)], dstbuf)
        pltpu.async_copy(x_hbm.at[srcbuf.at[0]], rows0, gs0)

        def body(i, _):
            for b in range(2):
                c = 2 * i + b
                pltpu.make_async_copy(x_hbm.at[srcbuf.at[c]], rows[b],
                                      gs[b]).wait()
                pltpu.async_copy(rows[b], shared.at[dstbuf.at[c]], ss[b],
                                 add=True)
                @pl.when(c >= 1)
                def _():
                    pltpu.make_async_copy(rows[1 - b],
                                          shared.at[dstbuf.at[c]],
                                          ss[1 - b]).wait()

                @pl.when(c + 1 < Q)
                def _():
                    pltpu.async_copy(x_hbm.at[srcbuf.at[c + 1]], rows[1 - b],
                                     gs[1 - b])
            return 0

        lax.fori_loop(0, Q // 2, body, 0)
        pltpu.make_async_copy(rows1, shared.at[dstbuf.at[0]], ss1).wait()
    plsc.subcore_barrier()

    _copy_out_tile_slice(shared, out0, sid)


def _sc_agg(x, src2d, dst2d):
    mesh = plsc.VectorSubcoreMesh(core_axis_name="c", subcore_axis_name="s",
                                  num_cores=1)
    f = pl.kernel(
        _sc_agg_body,
        out_type=jax.ShapeDtypeStruct((N, D), jnp.float32),
        mesh=mesh,
        scratch_types=[
            pltpu.VMEM_SHARED((ACC, D), jnp.float32),
            pltpu.VMEM((Q, CHUNK), jnp.int32),
            pltpu.VMEM((Q, CHUNK), jnp.int32),
            pltpu.VMEM((CHUNK, D), jnp.float32),
            pltpu.VMEM((CHUNK, D), jnp.float32),
            pltpu.SemaphoreType.DMA,
            pltpu.SemaphoreType.DMA,
            pltpu.SemaphoreType.DMA,
            pltpu.SemaphoreType.DMA,
        ],
    )
    return f(x, src2d, dst2d)


def _sc_cnt_body(dst_hbm, out0, shared, dstbuf, rows, ones, ssem):
    sid = lax.axis_index("s")
    half = CPW // 2

    _zero_tile_slice(shared, rows, sid)
    one = jnp.ones((16,), jnp.float32)

    def orow(i, _):
        for k in range(D // 16):
            ones[i, pl.ds(k * 16, 16)] = one
        return 0

    lax.fori_loop(0, CHUNK, orow, 0)
    plsc.subcore_barrier()

    def group(g, _):
        for k in range(8):
            pltpu.async_copy(ones, shared.at[dstbuf.at[8 * g + k]], ssem,
                             add=True)
        for k in range(8):
            pltpu.make_async_copy(ones, shared.at[dstbuf.at[8 * g]],
                                  ssem).wait()
        return 0

    for h in range(2):
        base = sid * CPW + h * half
        pltpu.sync_copy(dst_hbm.at[pl.ds(base, half)], dstbuf)
        lax.fori_loop(0, half // 8, group, 0)
    plsc.subcore_barrier()

    _copy_out_tile_slice(shared, out0, sid)


def _sc_cnt(dst2d):
    mesh = plsc.VectorSubcoreMesh(core_axis_name="c", subcore_axis_name="s",
                                  num_cores=1)
    f = pl.kernel(
        _sc_cnt_body,
        out_type=jax.ShapeDtypeStruct((N, D), jnp.float32),
        mesh=mesh,
        scratch_types=[
            pltpu.VMEM_SHARED((ACC, D), jnp.float32),
            pltpu.VMEM((CPW // 2, CHUNK), jnp.int32),
            pltpu.VMEM((CHUNK, D), jnp.float32),
            pltpu.VMEM((CHUNK, D), jnp.float32),
            pltpu.SemaphoreType.DMA,
        ],
    )
    return f(dst2d)


def _enc_body(x_ref, w0, b0, w1, b1, w2, b2, o_ref):
    h = jnp.maximum(jnp.dot(x_ref[...], w0[...],
                            preferred_element_type=jnp.float32) + b0[...], 0.0)
    h = jnp.maximum(jnp.dot(h, w1[...],
                            preferred_element_type=jnp.float32) + b1[...], 0.0)
    o_ref[...] = jnp.maximum(jnp.dot(h, w2[...],
                                     preferred_element_type=jnp.float32) + b2[...], 0.0)


def _tc_encoder(x, W0, b0, W1, b1, W2, b2):
    H = W0.shape[1]
    full = lambda s: pl.BlockSpec(s, lambda i: (0, 0))
    return pl.pallas_call(
        _enc_body,
        grid=(GRID,),
        in_specs=[
            pl.BlockSpec((BLK, D), lambda i: (i, 0)),
            full((D, H)), full((1, H)),
            full((H, H)), full((1, H)),
            full((H, D)), full((1, D)),
        ],
        out_specs=pl.BlockSpec((BLK, D), lambda i: (i, 0)),
        out_shape=jax.ShapeDtypeStruct((N, D), jnp.float32),
    )(x, W0, b0.reshape(1, -1), W1, b1.reshape(1, -1), W2, b2.reshape(1, -1))


def _conv_body(x_ref, p_ref, c_ref, wl, wr, b, o_ref):
    inv = 1.0 / jnp.maximum(c_ref[:, 0:1], 1.0)
    agg = p_ref[...] * inv
    o_ref[...] = jnp.maximum(
        jnp.dot(agg, wl[...], preferred_element_type=jnp.float32) + b[...]
        + jnp.dot(x_ref[...], wr[...], preferred_element_type=jnp.float32),
        0.0)


def _tc_conv(x, p, cnt, Wl, Wr, b):
    full = lambda s: pl.BlockSpec(s, lambda i: (0, 0))
    blk = lambda s: pl.BlockSpec(s, lambda i: (i, 0))
    return pl.pallas_call(
        _conv_body,
        grid=(GRID,),
        in_specs=[
            blk((BLK, D)), blk((BLK, D)), blk((BLK, D)),
            full((D, D)), full((D, D)), full((1, D)),
        ],
        out_specs=pl.BlockSpec((BLK, D), lambda i: (i, 0)),
        out_shape=jax.ShapeDtypeStruct((N, D), jnp.float32),
    )(x, p, cnt, Wl, Wr, b.reshape(1, -1))


def _ro_body(x_ref, ix_ref, w0a, w0b, b0, w1, b1, w2, b2, o_ref):
    h = jnp.maximum(
        jnp.dot(x_ref[...], w0a[...], preferred_element_type=jnp.float32)
        + jnp.dot(ix_ref[...], w0b[...], preferred_element_type=jnp.float32)
        + b0[...], 0.0)
    h = jnp.maximum(jnp.dot(h, w1[...],
                            preferred_element_type=jnp.float32) + b1[...], 0.0)
    o_ref[...] = jnp.dot(h, w2[...],
                         preferred_element_type=jnp.float32) + b2[...]


def _tc_readout(x, in_x, W0, b0, W1, b1, W2, b2):
    HR = W1.shape[0]
    full = lambda s: pl.BlockSpec(s, lambda i: (0, 0))
    blk = lambda s: pl.BlockSpec(s, lambda i: (i, 0))
    return pl.pallas_call(
        _ro_body,
        grid=(GRID,),
        in_specs=[
            blk((BLK, D)), blk((BLK, D)),
            full((D, HR)), full((D, HR)), full((1, HR)),
            full((HR, HR)), full((1, HR)),
            full((HR, D)), full((1, D)),
        ],
        out_specs=pl.BlockSpec((BLK, D), lambda i: (i, 0)),
        out_shape=jax.ShapeDtypeStruct((N, D), jnp.float32),
    )(x, in_x, W0[:D], W0[D:], b0.reshape(1, -1), W1, b1.reshape(1, -1),
      W2, b2.reshape(1, -1))


def kernel(in_x, edge_index, emb_W0, emb_b0, emb_W1, emb_b1, emb_W2, emb_b2,
           conv0_Wl, conv0_Wr, conv0_b, conv1_Wl, conv1_Wr, conv1_b,
           conv2_Wl, conv2_Wr, conv2_b, ro_W0, ro_b0, ro_W1, ro_b1,
           ro_W2, ro_b2):
    src = edge_index[0]
    dst = edge_index[1]
    npad = E_PAD - E
    src2d = jnp.concatenate(
        [src, jnp.zeros((npad,), jnp.int32)]).reshape(-1, CHUNK)
    dst2d = jnp.concatenate(
        [dst, jnp.full((npad,), N, jnp.int32)]).reshape(-1, CHUNK)

    cnt = _sc_cnt(dst2d)
    x = _tc_encoder(in_x, emb_W0, emb_b0, emb_W1, emb_b1, emb_W2, emb_b2)
    for Wl, Wr, b in ((conv0_Wl, conv0_Wr, conv0_b),
                      (conv1_Wl, conv1_Wr, conv1_b),
                      (conv2_Wl, conv2_Wr, conv2_b)):
        p = _sc_agg(x, src2d, dst2d)
        x = _tc_conv(x, p, cnt, Wl, Wr, b)
    return _tc_readout(x, in_x, ro_W0, ro_b0, ro_W1, ro_b1, ro_W2, ro_b2)
